# baseline (speedup 1.0000x reference)
import functools

import jax
import jax.numpy as jnp
from jax import lax
from jax.experimental import pallas as pl
from jax.experimental.pallas import tpu as pltpu
from jax.experimental.pallas import tpu_sc as plsc

NC = 2
NS = 16
LN = 16
CH = 128

N_NODES = 50000
ROWS_PER_TILE = 3200
NPAD = ROWS_PER_TILE * NS
E_EDGES = 800000
EDGES_PER_TILE = 50560
EPAD = EDGES_PER_TILE * NS
N_RCHUNKS = ROWS_PER_TILE // CH
KD = 5
NGD = 79
CHS = 64
KS = 5
NGS = 79

@functools.cache
def _get_mesh():
  return plsc.VectorSubcoreMesh(core_axis_name="c", subcore_axis_name="s")


def _zero_chunk(buf, w):
  z = jnp.zeros((LN,), jnp.float32)
  def body(r, _):
    for h in range(w // LN):
      buf[r, pl.ds(h * LN, LN)] = z
    return 0
  lax.fori_loop(0, CH, body, 0)


def _load_idx(hbm, off, dst_row):
  pltpu.sync_copy(hbm.at[pl.ds(off, CH)], dst_row.at[0])


def _deg_body(idx_hbm, out_hbm, acc, idxb, ones, pbuf, seml, sems, s):
  def zb(k, _):
    pltpu.sync_copy(ones.at[pl.ds(CH, CH)],
                    acc.at[pl.ds(s * ROWS_PER_TILE + k * CH, CH)])
    return 0
  lax.fori_loop(0, N_RCHUNKS, zb, 0)
  plsc.subcore_barrier()

  base = s * EDGES_PER_TILE
  def eb(g, _):
    goff = base + g * (KD * CH)
    lh = [pltpu.async_copy(idx_hbm.at[pl.ds(goff + k * CH, CH)],
                           idxb.at[k], seml) for k in range(KD)]
    for h in lh:
      h.wait()
    sh = [pltpu.async_copy(ones.at[pl.ds(0, CH)], acc.at[idxb.at[k]],
                           sems, add=True) for k in range(KD)]
    for h in sh:
      h.wait()
    return 0
  lax.fori_loop(0, NGD, eb, 0)
  plsc.subcore_barrier()

  def post(k, _):
    r0 = s * ROWS_PER_TILE + k * CH
    pltpu.sync_copy(acc.at[pl.ds(r0, CH)], pbuf)
    for h in range(CH // LN):
      v = pbuf[pl.ds(h * LN, LN)]
      pbuf[pl.ds(h * LN, LN)] = jnp.where(v > 0.0, 1.0 / v, 0.0)
    pltpu.sync_copy(pbuf, out_hbm.at[pl.ds(r0, CH)])
    return 0
  lax.fori_loop(0, N_RCHUNKS, post, 0)


def _deg_kernel(ni_hbm, ei_hbm, dinv_hbm, binv_hbm, acc, idxb, ones, pbuf,
                seml, sems):
  c = lax.axis_index("c")
  s = lax.axis_index("s")
  one = jnp.ones((LN,), jnp.float32)
  zero = jnp.zeros((LN,), jnp.float32)
  for h in range(CH // LN):
    ones[pl.ds(h * LN, LN)] = one
    ones[pl.ds(CH + h * LN, LN)] = zero

  @pl.when(c == 0)
  def _():
    _deg_body(ni_hbm, dinv_hbm, acc, idxb, ones, pbuf, seml, sems, s)

  @pl.when(c == 1)
  def _():
    _deg_body(ei_hbm, binv_hbm, acc, idxb, ones, pbuf, seml, sems, s)


@jax.jit
def _degrees(ni_p, ei_p):
  f = pl.kernel(
      _deg_kernel,
      out_type=[jax.ShapeDtypeStruct((NPAD,), jnp.float32),
                jax.ShapeDtypeStruct((NPAD,), jnp.float32)],
      mesh=_get_mesh(),
      compiler_params=pltpu.CompilerParams(use_tc_tiling_on_sc=False),
      scratch_types=[
          pltpu.VMEM_SHARED((NPAD,), jnp.float32),
          pltpu.VMEM((KD, CH), jnp.int32),
          pltpu.VMEM((2 * CH,), jnp.float32),
          pltpu.VMEM((CH,), jnp.float32),
          pltpu.SemaphoreType.DMA,
          pltpu.SemaphoreType.DMA,
      ],
  )
  return f(ni_p, ei_p)


def _spmm_kernel(w, relu, table_hbm, src_hbm, dst_hbm, inv_hbm, out_hbm,
                 acc, sidx0, sidx1, didx0, didx1, shft0, shft1, rows0, rows1,
                 pbuf, ibuf, seml0, semls1, seml1, semg, sems):
  c = lax.axis_index("c")
  s = lax.axis_index("s")
  shift = c * NPAD

  _zero_chunk(pbuf, w)
  def zrow(k, _):
    pltpu.sync_copy(pbuf, acc.at[pl.ds(s * ROWS_PER_TILE + k * CH, CH), :])
    return 0
  lax.fori_loop(0, N_RCHUNKS, zrow, 0)
  plsc.subcore_barrier()

  def _shifts(sidx, shft):
    for k in range(KS):
      for h in range(CHS // LN):
        shft[k, pl.ds(h * LN, LN)] = (
            sidx[pl.ds(k * CHS + h * LN, LN)] + shift)

  base = s * EDGES_PER_TILE
  GB = KS * CHS
  def eb(i, _):
    off0 = base + (2 * i) * GB
    off1 = off0 + GB
    pltpu.sync_copy(src_hbm.at[pl.ds(off0, GB)], sidx0)
    l0 = [pltpu.async_copy(dst_hbm.at[pl.ds(off0 + k * CHS, CHS)],
                           didx0.at[k], seml0) for k in range(KS)]
    _shifts(sidx0, shft0)
    g0 = [pltpu.async_copy(table_hbm.at[shft0.at[k]], rows0.at[k], semg)
          for k in range(KS)]
    s1 = pltpu.async_copy(src_hbm.at[pl.ds(off1, GB)], sidx1, semls1)
    l1 = [pltpu.async_copy(dst_hbm.at[pl.ds(off1 + k * CHS, CHS)],
                           didx1.at[k], seml1) for k in range(KS)]
    for h in l0:
      h.wait()
    for h in g0:
      h.wait()
    sc0 = [pltpu.async_copy(rows0.at[k], acc.at[didx0.at[k]], sems, add=True)
           for k in range(KS)]
    s1.wait()
    _shifts(sidx1, shft1)
    g1 = [pltpu.async_copy(table_hbm.at[shft1.at[k]], rows1.at[k], semg)
          for k in range(KS)]
    for h in l1:
      h.wait()
    for h in sc0:
      h.wait()
    for h in g1:
      h.wait()
    sc1 = [pltpu.async_copy(rows1.at[k], acc.at[didx1.at[k]], sems, add=True)
           for k in range(KS)]
    for h in sc1:
      h.wait()
    return 0
  lax.fori_loop(0, NGS, eb, 0)
  plsc.subcore_barrier()

  def post(k, _):
    r0 = s * ROWS_PER_TILE + k * CH
    pltpu.sync_copy(acc.at[pl.ds(r0, CH), :], pbuf)
    pltpu.sync_copy(inv_hbm.at[pl.ds(r0, CH)], ibuf.at[pl.ds(0, CH)])
    def prow(r, _):
      sv = ibuf[pl.ds(r, LN)][0]
      for h in range(w // LN):
        v = pbuf[r, pl.ds(h * LN, LN)] * sv
        if relu:
          v = jnp.maximum(v, 0.0)
        pbuf[r, pl.ds(h * LN, LN)] = v
      return 0
    lax.fori_loop(0, CH, prow, 0)
    pltpu.sync_copy(pbuf, out_hbm.at[pl.ds(shift + r0, CH), :])
    return 0
  lax.fori_loop(0, N_RCHUNKS, post, 0)


@functools.partial(jax.jit, static_argnums=(4, 5))
def _spmm(table, src, dst, inv, w, relu):
  f = pl.kernel(
      functools.partial(_spmm_kernel, w, relu),
      out_type=jax.ShapeDtypeStruct((2 * NPAD, w), jnp.float32),
      mesh=_get_mesh(),
      compiler_params=pltpu.CompilerParams(use_tc_tiling_on_sc=False),
      scratch_types=[
          pltpu.VMEM_SHARED((NPAD, w), jnp.float32),
          pltpu.VMEM((KS * CHS,), jnp.int32),
          pltpu.VMEM((KS * CHS,), jnp.int32),
          pltpu.VMEM((KS, CHS), jnp.int32),
          pltpu.VMEM((KS, CHS), jnp.int32),
          pltpu.VMEM((KS, CHS), jnp.int32),
          pltpu.VMEM((KS, CHS), jnp.int32),
          pltpu.VMEM((KS, CHS, w), jnp.float32),
          pltpu.VMEM((KS, CHS, w), jnp.float32),
          pltpu.VMEM((CH, w), jnp.float32),
          pltpu.VMEM((CH + LN,), jnp.float32),
          pltpu.SemaphoreType.DMA,
          pltpu.SemaphoreType.DMA,
          pltpu.SemaphoreType.DMA,
          pltpu.SemaphoreType.DMA,
          pltpu.SemaphoreType.DMA,
      ],
  )
  return f(table, src, dst, inv)


def _head_kernel(nt0, nt1, nt2, nt3, et0, et1, et2, et3,
                 marks_hbm, emarks_hbm, h_hbm,
                 mb, eb, shft, g32, g16a, g16b, hbuf):
  c = lax.axis_index("c")
  s = lax.axis_index("s")
  wid = c * NS + s
  r0 = wid * CH
  nts = [nt0, nt1, nt2, nt3]
  ets = [et0, et1, et2, et3]

  _load_idx(marks_hbm, r0, mb)
  _load_idx(emarks_hbm, r0, eb)

  for l in range(4):
    for half in range(2):
      shift = half * NPAD
      for h in range(CH // LN):
        shft[0, pl.ds(h * LN, LN)] = mb[0, pl.ds(h * LN, LN)] + shift
      pltpu.sync_copy(nts[l].at[shft.at[0]], g32)
      col0 = 256 + l * 64 + half * 32
      def cpy(r, _, col0=col0):
        for h2 in range(2):
          hbuf[r, pl.ds(col0 + h2 * LN, LN)] = g32[r, pl.ds(h2 * LN, LN)]
        return 0
      lax.fori_loop(0, CH, cpy, 0)

  for l in range(4):
    for half in range(2):
      shift = half * NPAD
      for h in range(CH // LN):
        shft[0, pl.ds(h * LN, LN)] = eb[0, pl.ds(h * LN, LN)] + shift
      pltpu.sync_copy(ets[l].at[shft.at[0]], g16a)
      for h in range(CH // LN):
        shft[0, pl.ds(h * LN, LN)] = eb[0, pl.ds(h * LN, LN)] + (shift + 1)
      pltpu.sync_copy(ets[l].at[shft.at[0]], g16b)
      cmin = l * 32 + half * 16
      def mm(r, _, cmin=cmin):
        v1 = g16a[r, pl.ds(0, LN)]
        v2 = g16b[r, pl.ds(0, LN)]
        hbuf[r, pl.ds(cmin, LN)] = jnp.minimum(v1, v2)
        hbuf[r, pl.ds(128 + cmin, LN)] = jnp.maximum(v1, v2)
        return 0
      lax.fori_loop(0, CH, mm, 0)

  pltpu.sync_copy(hbuf, h_hbm.at[pl.ds(r0, CH), :])


@jax.jit
def _head_gather(nts, ets, marks, emarks):
  f = pl.kernel(
      _head_kernel,
      out_type=jax.ShapeDtypeStruct((4096, 512), jnp.float32),
      mesh=_get_mesh(),
      compiler_params=pltpu.CompilerParams(use_tc_tiling_on_sc=False),
      scratch_types=[
          pltpu.VMEM((1, CH), jnp.int32),
          pltpu.VMEM((1, CH), jnp.int32),
          pltpu.VMEM((1, CH), jnp.int32),
          pltpu.VMEM((CH, 32), jnp.float32),
          pltpu.VMEM((CH, 16), jnp.float32),
          pltpu.VMEM((CH, 16), jnp.float32),
          pltpu.VMEM((CH, 512), jnp.float32),
      ],
  )
  return f(*nts, *ets, marks, emarks)


BL = 512


def _mm_body(nparts, widths, *refs):
  parts = refs[:nparts]
  w_ref = refs[nparts]
  b_ref = refs[nparts + 1]
  o_ref = refs[nparts + 2]
  acc = jnp.zeros(o_ref.shape[1:], jnp.float32)
  off = 0
  for p, wp in zip(parts, widths):
    acc = acc + jnp.dot(p[...], w_ref[0, off:off + wp, :],
                        preferred_element_type=jnp.float32)
    off += wp
  o_ref[...] = (acc + b_ref[0])[None]


@functools.partial(jax.jit, static_argnums=(3,))
def _tc_mm(parts, W, b, half):
  nparts = len(parts)
  widths = tuple(p.shape[1] for p in parts)
  din = sum(widths)
  W2 = W.reshape(din, 2, half).transpose(1, 0, 2)
  b2 = b.reshape(2, 1, half)
  in_specs = [pl.BlockSpec((BL, wp), lambda i, c: (i, 0)) for wp in widths]
  in_specs.append(pl.BlockSpec((1, din, half), lambda i, c: (c, 0, 0)))
  in_specs.append(pl.BlockSpec((1, 1, half), lambda i, c: (c, 0, 0)))
  out = pl.pallas_call(
      functools.partial(_mm_body, nparts, widths),
      grid=(NPAD // BL, 2),
      in_specs=in_specs,
      out_specs=pl.BlockSpec((1, BL, half), lambda i, c: (c, i, 0)),
      out_shape=jax.ShapeDtypeStruct((2, NPAD, half), jnp.float32),
  )(*parts, W2, b2)
  return out.reshape(2 * NPAD, half)


def _mlp_body(h_ref, w1_ref, b1_ref, w2_ref, b2_ref, o_ref):
  h1 = jnp.maximum(
      jnp.dot(h_ref[...], w1_ref[...], preferred_element_type=jnp.float32)
      + b1_ref[...], 0.0)
  z = jnp.dot(h1, w2_ref[...], preferred_element_type=jnp.float32) + b2_ref[...]
  m = jnp.max(z, axis=1, keepdims=True)
  lse = m + jnp.log(jnp.sum(jnp.exp(z - m), axis=1, keepdims=True))
  o_ref[...] = z - lse


@jax.jit
def _tc_head(h, W1, b1, W2, b2):
  HB = 512
  return pl.pallas_call(
      _mlp_body,
      grid=(4096 // HB,),
      in_specs=[
          pl.BlockSpec((HB, 512), lambda i: (i, 0)),
          pl.BlockSpec((512, 128), lambda i: (0, 0)),
          pl.BlockSpec((1, 128), lambda i: (0, 0)),
          pl.BlockSpec((128, 2), lambda i: (0, 0)),
          pl.BlockSpec((1, 2), lambda i: (0, 0)),
      ],
      out_specs=pl.BlockSpec((HB, 2), lambda i: (i, 0)),
      out_shape=jax.ShapeDtypeStruct((4096, 2), jnp.float32),
  )(h, W1, b1.reshape(1, -1), W2, b2.reshape(1, -1))


def kernel(x, edge_index, marks, edge_x, edge_marks,
           Wn0, bn0, We0, be0, Wn1, bn1, We1, be1,
           Wn2, bn2, We2, be2, Wn3, bn3, We3, be3,
           W1, b1, W2, b2):
  pad_idx = jnp.full((EPAD - E_EDGES,), NPAD - 1, jnp.int32)
  ni_p = jnp.concatenate([edge_index[0], pad_idx])
  ei_p = jnp.concatenate([edge_index[1], pad_idx])

  dinv, binv = _degrees(ni_p, ei_p)

  x_p = jnp.zeros((NPAD, x.shape[1]), jnp.float32).at[:N_NODES].set(x)
  ex_p = jnp.zeros((NPAD, edge_x.shape[1]), jnp.float32).at[:N_NODES].set(edge_x)

  cur_parts = [x_p]
  cure_parts = [ex_p]
  node_outs, edge_outs = [], []
  Wns = [(Wn0, bn0), (Wn1, bn1), (Wn2, bn2), (Wn3, bn3)]
  Wes = [(We0, be0), (We1, be1), (We2, be2), (We3, be3)]
  for (Wn, bn), (We, be) in zip(Wns, Wes):
    xw = _tc_mm(cur_parts, Wn, bn, 32)
    ew = _tc_mm(cure_parts, We, be, 16)
    t = _spmm(xw, ni_p, ei_p, binv, 32, False)
    nout = _spmm(t, ei_p, ni_p, dinv, 32, True)
    t2 = _spmm(ew, ei_p, ni_p, dinv, 16, False)
    eout = _spmm(t2, ni_p, ei_p, binv, 16, True)
    node_outs.append(nout)
    edge_outs.append(eout)
    n3 = nout.reshape(2, NPAD, 32)
    e3 = eout.reshape(2, NPAD, 16)
    cur_parts = [n3[0], n3[1]]
    cure_parts = [e3[0], e3[1]]

  h = _head_gather(node_outs, edge_outs, marks, edge_marks)
  return _tc_head(h, W1, b1, W2, b2)


# speedup vs baseline: 1.0652x; 1.0652x over previous
import functools

import jax
import jax.numpy as jnp
from jax import lax
from jax.experimental import pallas as pl
from jax.experimental.pallas import tpu as pltpu
from jax.experimental.pallas import tpu_sc as plsc

NC = 2
NS = 16
LN = 16
CH = 128

N_NODES = 50000
ROWS_PER_TILE = 3200
NPAD = ROWS_PER_TILE * NS
E_EDGES = 800000
EDGES_PER_TILE = 50560
EPAD = EDGES_PER_TILE * NS
N_RCHUNKS = ROWS_PER_TILE // CH
CROWS_PER_TILE = EDGES_PER_TILE // CH
KD = 5
NGD = 79
KS = 5
NGS = 79

@functools.cache
def _get_mesh():
  return plsc.VectorSubcoreMesh(core_axis_name="c", subcore_axis_name="s")


def _zero_chunk(buf, w):
  z = jnp.zeros((LN,), jnp.float32)
  def body(r, _):
    for h in range(w // LN):
      buf[r, pl.ds(h * LN, LN)] = z
    return 0
  lax.fori_loop(0, CH, body, 0)


def _load_idx(hbm, off, dst_row):
  pltpu.sync_copy(hbm.at[pl.ds(off, CH)], dst_row.at[0])


def _deg_body(idx_hbm, out_hbm, acc, idxb, ones, pbuf, seml, sems, s):
  def zb(k, _):
    pltpu.sync_copy(ones.at[pl.ds(CH, CH)],
                    acc.at[pl.ds(s * ROWS_PER_TILE + k * CH, CH)])
    return 0
  lax.fori_loop(0, N_RCHUNKS, zb, 0)
  plsc.subcore_barrier()

  base = s * CROWS_PER_TILE
  def eb(g, _):
    row0 = base + g * KD
    pltpu.sync_copy(idx_hbm.at[pl.ds(row0, KD), :], idxb)
    sh = [pltpu.async_copy(ones.at[pl.ds(0, CH)], acc.at[idxb.at[k]],
                           sems, add=True) for k in range(KD)]
    for h in sh:
      h.wait()
    return 0
  lax.fori_loop(0, NGD, eb, 0)
  plsc.subcore_barrier()

  def post(k, _):
    r0 = s * ROWS_PER_TILE + k * CH
    pltpu.sync_copy(acc.at[pl.ds(r0, CH)], pbuf)
    for h in range(CH // LN):
      v = pbuf[pl.ds(h * LN, LN)]
      pbuf[pl.ds(h * LN, LN)] = jnp.where(v > 0.0, 1.0 / v, 0.0)
    pltpu.sync_copy(pbuf, out_hbm.at[pl.ds(r0, CH)])
    return 0
  lax.fori_loop(0, N_RCHUNKS, post, 0)


def _deg_kernel(ni_hbm, ei_hbm, dinv_hbm, binv_hbm, acc, idxb, ones, pbuf,
                seml, sems):
  c = lax.axis_index("c")
  s = lax.axis_index("s")
  one = jnp.ones((LN,), jnp.float32)
  zero = jnp.zeros((LN,), jnp.float32)
  for h in range(CH // LN):
    ones[pl.ds(h * LN, LN)] = one
    ones[pl.ds(CH + h * LN, LN)] = zero

  @pl.when(c == 0)
  def _():
    _deg_body(ni_hbm, dinv_hbm, acc, idxb, ones, pbuf, seml, sems, s)

  @pl.when(c == 1)
  def _():
    _deg_body(ei_hbm, binv_hbm, acc, idxb, ones, pbuf, seml, sems, s)


@jax.jit
def _degrees(ni_p, ei_p):
  f = pl.kernel(
      _deg_kernel,
      out_type=[jax.ShapeDtypeStruct((NPAD,), jnp.float32),
                jax.ShapeDtypeStruct((NPAD,), jnp.float32)],
      mesh=_get_mesh(),
      compiler_params=pltpu.CompilerParams(use_tc_tiling_on_sc=False),
      scratch_types=[
          pltpu.VMEM_SHARED((NPAD,), jnp.float32),
          pltpu.VMEM((KD, CH), jnp.int32),
          pltpu.VMEM((2 * CH,), jnp.float32),
          pltpu.VMEM((CH,), jnp.float32),
          pltpu.SemaphoreType.DMA,
          pltpu.SemaphoreType.DMA,
      ],
  )
  return f(ni_p, ei_p)


def _spmm_kernel(w, relu, table_hbm, src_hbm, dst_hbm, inv_hbm, out_hbm,
                 acc, sidx, didx, shft, rows, pbuf, ibuf, seml, semg, sems):
  c = lax.axis_index("c")
  s = lax.axis_index("s")
  shift = c * NPAD

  _zero_chunk(pbuf, w)
  def zrow(k, _):
    pltpu.sync_copy(pbuf, acc.at[pl.ds(s * ROWS_PER_TILE + k * CH, CH), :])
    return 0
  lax.fori_loop(0, N_RCHUNKS, zrow, 0)
  plsc.subcore_barrier()

  base = s * CROWS_PER_TILE
  def eb(g, _):
    row0 = base + g * KS
    pltpu.sync_copy(src_hbm.at[pl.ds(row0, KS), :], sidx)
    lh = pltpu.async_copy(dst_hbm.at[pl.ds(row0, KS), :], didx, seml)
    for k in range(KS):
      for h in range(CH // LN):
        shft[k, pl.ds(h * LN, LN)] = sidx[k, pl.ds(h * LN, LN)] + shift
    gh = [pltpu.async_copy(table_hbm.at[shft.at[k]], rows.at[k], semg)
          for k in range(KS)]
    lh.wait()
    for h in gh:
      h.wait()
    sh = [pltpu.async_copy(rows.at[k], acc.at[didx.at[k]], sems, add=True)
          for k in range(KS)]
    for h in sh:
      h.wait()
    return 0
  lax.fori_loop(0, NGS, eb, 0)
  plsc.subcore_barrier()

  def post(k, _):
    r0 = s * ROWS_PER_TILE + k * CH
    pltpu.sync_copy(acc.at[pl.ds(r0, CH), :], pbuf)
    pltpu.sync_copy(inv_hbm.at[pl.ds(r0, CH)], ibuf.at[pl.ds(0, CH)])
    def prow(r, _):
      sv = ibuf[pl.ds(r, LN)][0]
      for h in range(w // LN):
        v = pbuf[r, pl.ds(h * LN, LN)] * sv
        if relu:
          v = jnp.maximum(v, 0.0)
        pbuf[r, pl.ds(h * LN, LN)] = v
      return 0
    lax.fori_loop(0, CH, prow, 0)
    pltpu.sync_copy(pbuf, out_hbm.at[pl.ds(shift + r0, CH), :])
    return 0
  lax.fori_loop(0, N_RCHUNKS, post, 0)


@functools.partial(jax.jit, static_argnums=(4, 5))
def _spmm(table, src, dst, inv, w, relu):
  f = pl.kernel(
      functools.partial(_spmm_kernel, w, relu),
      out_type=jax.ShapeDtypeStruct((2 * NPAD, w), jnp.float32),
      mesh=_get_mesh(),
      compiler_params=pltpu.CompilerParams(use_tc_tiling_on_sc=False),
      scratch_types=[
          pltpu.VMEM_SHARED((NPAD, w), jnp.float32),
          pltpu.VMEM((KS, CH), jnp.int32),
          pltpu.VMEM((KS, CH), jnp.int32),
          pltpu.VMEM((KS, CH), jnp.int32),
          pltpu.VMEM((KS, CH, w), jnp.float32),
          pltpu.VMEM((CH, w), jnp.float32),
          pltpu.VMEM((CH + LN,), jnp.float32),
          pltpu.SemaphoreType.DMA,
          pltpu.SemaphoreType.DMA,
          pltpu.SemaphoreType.DMA,
      ],
  )
  return f(table, src, dst, inv)


def _head_kernel(nt0, nt1, nt2, nt3, et0, et1, et2, et3,
                 marks_hbm, emarks_hbm, h_hbm,
                 mb, eb, shft, g32, g16a, g16b, hbuf):
  c = lax.axis_index("c")
  s = lax.axis_index("s")
  wid = c * NS + s
  r0 = wid * CH
  nts = [nt0, nt1, nt2, nt3]
  ets = [et0, et1, et2, et3]

  _load_idx(marks_hbm, r0, mb)
  _load_idx(emarks_hbm, r0, eb)

  for l in range(4):
    for half in range(2):
      shift = half * NPAD
      for h in range(CH // LN):
        shft[0, pl.ds(h * LN, LN)] = mb[0, pl.ds(h * LN, LN)] + shift
      pltpu.sync_copy(nts[l].at[shft.at[0]], g32)
      col0 = 256 + l * 64 + half * 32
      def cpy(r, _, col0=col0):
        for h2 in range(2):
          hbuf[r, pl.ds(col0 + h2 * LN, LN)] = g32[r, pl.ds(h2 * LN, LN)]
        return 0
      lax.fori_loop(0, CH, cpy, 0)

  for l in range(4):
    for half in range(2):
      shift = half * NPAD
      for h in range(CH // LN):
        shft[0, pl.ds(h * LN, LN)] = eb[0, pl.ds(h * LN, LN)] + shift
      pltpu.sync_copy(ets[l].at[shft.at[0]], g16a)
      for h in range(CH // LN):
        shft[0, pl.ds(h * LN, LN)] = eb[0, pl.ds(h * LN, LN)] + (shift + 1)
      pltpu.sync_copy(ets[l].at[shft.at[0]], g16b)
      cmin = l * 32 + half * 16
      def mm(r, _, cmin=cmin):
        v1 = g16a[r, pl.ds(0, LN)]
        v2 = g16b[r, pl.ds(0, LN)]
        hbuf[r, pl.ds(cmin, LN)] = jnp.minimum(v1, v2)
        hbuf[r, pl.ds(128 + cmin, LN)] = jnp.maximum(v1, v2)
        return 0
      lax.fori_loop(0, CH, mm, 0)

  pltpu.sync_copy(hbuf, h_hbm.at[pl.ds(r0, CH), :])


@jax.jit
def _head_gather(nts, ets, marks, emarks):
  f = pl.kernel(
      _head_kernel,
      out_type=jax.ShapeDtypeStruct((4096, 512), jnp.float32),
      mesh=_get_mesh(),
      compiler_params=pltpu.CompilerParams(use_tc_tiling_on_sc=False),
      scratch_types=[
          pltpu.VMEM((1, CH), jnp.int32),
          pltpu.VMEM((1, CH), jnp.int32),
          pltpu.VMEM((1, CH), jnp.int32),
          pltpu.VMEM((CH, 32), jnp.float32),
          pltpu.VMEM((CH, 16), jnp.float32),
          pltpu.VMEM((CH, 16), jnp.float32),
          pltpu.VMEM((CH, 512), jnp.float32),
      ],
  )
  return f(*nts, *ets, marks, emarks)


BL = 512


def _mm_body(nparts, widths, *refs):
  parts = refs[:nparts]
  w_ref = refs[nparts]
  b_ref = refs[nparts + 1]
  o_ref = refs[nparts + 2]
  acc = jnp.zeros(o_ref.shape[1:], jnp.float32)
  off = 0
  for p, wp in zip(parts, widths):
    acc = acc + jnp.dot(p[...], w_ref[0, off:off + wp, :],
                        preferred_element_type=jnp.float32)
    off += wp
  o_ref[...] = (acc + b_ref[0])[None]


@functools.partial(jax.jit, static_argnums=(3,))
def _tc_mm(parts, W, b, half):
  nparts = len(parts)
  widths = tuple(p.shape[1] for p in parts)
  din = sum(widths)
  W2 = W.reshape(din, 2, half).transpose(1, 0, 2)
  b2 = b.reshape(2, 1, half)
  in_specs = [pl.BlockSpec((BL, wp), lambda i, c: (i, 0)) for wp in widths]
  in_specs.append(pl.BlockSpec((1, din, half), lambda i, c: (c, 0, 0)))
  in_specs.append(pl.BlockSpec((1, 1, half), lambda i, c: (c, 0, 0)))
  out = pl.pallas_call(
      functools.partial(_mm_body, nparts, widths),
      grid=(NPAD // BL, 2),
      in_specs=in_specs,
      out_specs=pl.BlockSpec((1, BL, half), lambda i, c: (c, i, 0)),
      out_shape=jax.ShapeDtypeStruct((2, NPAD, half), jnp.float32),
  )(*parts, W2, b2)
  return out.reshape(2 * NPAD, half)


def _mlp_body(h_ref, w1_ref, b1_ref, w2_ref, b2_ref, o_ref):
  h1 = jnp.maximum(
      jnp.dot(h_ref[...], w1_ref[...], preferred_element_type=jnp.float32)
      + b1_ref[...], 0.0)
  z = jnp.dot(h1, w2_ref[...], preferred_element_type=jnp.float32) + b2_ref[...]
  m = jnp.max(z, axis=1, keepdims=True)
  lse = m + jnp.log(jnp.sum(jnp.exp(z - m), axis=1, keepdims=True))
  o_ref[...] = z - lse


@jax.jit
def _tc_head(h, W1, b1, W2, b2):
  HB = 512
  return pl.pallas_call(
      _mlp_body,
      grid=(4096 // HB,),
      in_specs=[
          pl.BlockSpec((HB, 512), lambda i: (i, 0)),
          pl.BlockSpec((512, 128), lambda i: (0, 0)),
          pl.BlockSpec((1, 128), lambda i: (0, 0)),
          pl.BlockSpec((128, 2), lambda i: (0, 0)),
          pl.BlockSpec((1, 2), lambda i: (0, 0)),
      ],
      out_specs=pl.BlockSpec((HB, 2), lambda i: (i, 0)),
      out_shape=jax.ShapeDtypeStruct((4096, 2), jnp.float32),
  )(h, W1, b1.reshape(1, -1), W2, b2.reshape(1, -1))


def kernel(x, edge_index, marks, edge_x, edge_marks,
           Wn0, bn0, We0, be0, Wn1, bn1, We1, be1,
           Wn2, bn2, We2, be2, Wn3, bn3, We3, be3,
           W1, b1, W2, b2):
  pad_idx = jnp.full((EPAD - E_EDGES,), NPAD - 1, jnp.int32)
  ni_p = jnp.concatenate([edge_index[0], pad_idx]).reshape(-1, CH)
  ei_p = jnp.concatenate([edge_index[1], pad_idx]).reshape(-1, CH)

  dinv, binv = _degrees(ni_p, ei_p)

  x_p = jnp.zeros((NPAD, x.shape[1]), jnp.float32).at[:N_NODES].set(x)
  ex_p = jnp.zeros((NPAD, edge_x.shape[1]), jnp.float32).at[:N_NODES].set(edge_x)

  cur_parts = [x_p]
  cure_parts = [ex_p]
  node_outs, edge_outs = [], []
  Wns = [(Wn0, bn0), (Wn1, bn1), (Wn2, bn2), (Wn3, bn3)]
  Wes = [(We0, be0), (We1, be1), (We2, be2), (We3, be3)]
  for (Wn, bn), (We, be) in zip(Wns, Wes):
    xw = _tc_mm(cur_parts, Wn, bn, 32)
    ew = _tc_mm(cure_parts, We, be, 16)
    t = _spmm(xw, ni_p, ei_p, binv, 32, False)
    nout = _spmm(t, ei_p, ni_p, dinv, 32, True)
    t2 = _spmm(ew, ei_p, ni_p, dinv, 16, False)
    eout = _spmm(t2, ni_p, ei_p, binv, 16, True)
    node_outs.append(nout)
    edge_outs.append(eout)
    n3 = nout.reshape(2, NPAD, 32)
    e3 = eout.reshape(2, NPAD, 16)
    cur_parts = [n3[0], n3[1]]
    cure_parts = [e3[0], e3[1]]

  h = _head_gather(node_outs, edge_outs, marks, edge_marks)
  return _tc_head(h, W1, b1, W2, b2)


# speedup vs baseline: 1.0965x; 1.0294x over previous
import functools

import jax
import jax.numpy as jnp
from jax import lax
from jax.experimental import pallas as pl
from jax.experimental.pallas import tpu as pltpu
from jax.experimental.pallas import tpu_sc as plsc

NC = 2
NS = 16
LN = 16
CH = 128

N_NODES = 50000
ROWS_PER_TILE = 3200
NPAD = ROWS_PER_TILE * NS
E_EDGES = 800000
EDGES_PER_TILE = 50688
EPAD = EDGES_PER_TILE * NS
N_RCHUNKS = ROWS_PER_TILE // CH
CROWS_PER_TILE = EDGES_PER_TILE // CH
KD = 6
NGD = 66

@functools.cache
def _get_mesh():
  return plsc.VectorSubcoreMesh(core_axis_name="c", subcore_axis_name="s")


def _zero_chunk(buf, w):
  z = jnp.zeros((LN,), jnp.float32)
  def body(r, _):
    for h in range(w // LN):
      buf[r, pl.ds(h * LN, LN)] = z
    return 0
  lax.fori_loop(0, CH, body, 0)


def _load_idx(hbm, off, dst_row):
  pltpu.sync_copy(hbm.at[pl.ds(off, CH)], dst_row.at[0])


def _deg_body(idx_hbm, out_hbm, acc, idxb, ones, pbuf, seml, sems, s):
  def zb(k, _):
    pltpu.sync_copy(ones.at[pl.ds(CH, CH)],
                    acc.at[pl.ds(s * ROWS_PER_TILE + k * CH, CH)])
    return 0
  lax.fori_loop(0, N_RCHUNKS, zb, 0)
  plsc.subcore_barrier()

  base = s * CROWS_PER_TILE
  def eb(g, _):
    row0 = base + g * KD
    pltpu.sync_copy(idx_hbm.at[pl.ds(row0, KD), :], idxb)
    sh = [pltpu.async_copy(ones.at[pl.ds(0, CH)], acc.at[idxb.at[k]],
                           sems, add=True) for k in range(KD)]
    for h in sh:
      h.wait()
    return 0
  lax.fori_loop(0, NGD, eb, 0)
  plsc.subcore_barrier()

  def post(k, _):
    r0 = s * ROWS_PER_TILE + k * CH
    pltpu.sync_copy(acc.at[pl.ds(r0, CH)], pbuf)
    for h in range(CH // LN):
      v = pbuf[pl.ds(h * LN, LN)]
      pbuf[pl.ds(h * LN, LN)] = jnp.where(v > 0.0, 1.0 / v, 0.0)
    pltpu.sync_copy(pbuf, out_hbm.at[pl.ds(r0, CH)])
    return 0
  lax.fori_loop(0, N_RCHUNKS, post, 0)


def _deg_kernel(ni_hbm, ei_hbm, dinv_hbm, binv_hbm, acc, idxb, ones, pbuf,
                seml, sems):
  c = lax.axis_index("c")
  s = lax.axis_index("s")
  one = jnp.ones((LN,), jnp.float32)
  zero = jnp.zeros((LN,), jnp.float32)
  for h in range(CH // LN):
    ones[pl.ds(h * LN, LN)] = one
    ones[pl.ds(CH + h * LN, LN)] = zero

  @pl.when(c == 0)
  def _():
    _deg_body(ni_hbm, dinv_hbm, acc, idxb, ones, pbuf, seml, sems, s)

  @pl.when(c == 1)
  def _():
    _deg_body(ei_hbm, binv_hbm, acc, idxb, ones, pbuf, seml, sems, s)


@jax.jit
def _degrees(ni_p, ei_p):
  f = pl.kernel(
      _deg_kernel,
      out_type=[jax.ShapeDtypeStruct((NPAD,), jnp.float32),
                jax.ShapeDtypeStruct((NPAD,), jnp.float32)],
      mesh=_get_mesh(),
      compiler_params=pltpu.CompilerParams(use_tc_tiling_on_sc=False),
      scratch_types=[
          pltpu.VMEM_SHARED((NPAD,), jnp.float32),
          pltpu.VMEM((KD, CH), jnp.int32),
          pltpu.VMEM((2 * CH,), jnp.float32),
          pltpu.VMEM((CH,), jnp.float32),
          pltpu.SemaphoreType.DMA,
          pltpu.SemaphoreType.DMA,
      ],
  )
  return f(ni_p, ei_p)


def _spmm_kernel(w, relu, ks, table_hbm, src_hbm, dst_hbm, inv_hbm, out_hbm,
                 acc, sidx, didx, shft, rows, ibuf, seml, semg, sems):
  c = lax.axis_index("c")
  s = lax.axis_index("s")
  shift = c * NPAD
  ngs = CROWS_PER_TILE // ks
  pbuf = rows.at[0]

  _zero_chunk(pbuf, w)
  def zrow(k, _):
    pltpu.sync_copy(pbuf, acc.at[pl.ds(s * ROWS_PER_TILE + k * CH, CH), :])
    return 0
  lax.fori_loop(0, N_RCHUNKS, zrow, 0)
  plsc.subcore_barrier()

  base = s * CROWS_PER_TILE
  def eb(g, _):
    row0 = base + g * ks
    pltpu.sync_copy(src_hbm.at[pl.ds(row0, ks), :], sidx)
    lh = pltpu.async_copy(dst_hbm.at[pl.ds(row0, ks), :], didx, seml)
    for k in range(ks):
      for h in range(CH // LN):
        shft[k, pl.ds(h * LN, LN)] = sidx[k, pl.ds(h * LN, LN)] + shift
    gh = [pltpu.async_copy(table_hbm.at[shft.at[k]], rows.at[k], semg)
          for k in range(ks)]
    lh.wait()
    for h in gh:
      h.wait()
    sh = [pltpu.async_copy(rows.at[k], acc.at[didx.at[k]], sems, add=True)
          for k in range(ks)]
    for h in sh:
      h.wait()
    return 0
  lax.fori_loop(0, ngs, eb, 0)
  plsc.subcore_barrier()

  def post(k, _):
    r0 = s * ROWS_PER_TILE + k * CH
    pltpu.sync_copy(acc.at[pl.ds(r0, CH), :], pbuf)
    pltpu.sync_copy(inv_hbm.at[pl.ds(r0, CH)], ibuf.at[pl.ds(0, CH)])
    def prow(r, _):
      sv = ibuf[pl.ds(r, LN)][0]
      for h in range(w // LN):
        v = pbuf[r, pl.ds(h * LN, LN)] * sv
        if relu:
          v = jnp.maximum(v, 0.0)
        pbuf[r, pl.ds(h * LN, LN)] = v
      return 0
    lax.fori_loop(0, CH, prow, 0)
    pltpu.sync_copy(pbuf, out_hbm.at[pl.ds(shift + r0, CH), :])
    return 0
  lax.fori_loop(0, N_RCHUNKS, post, 0)


@functools.partial(jax.jit, static_argnums=(4, 5))
def _spmm(table, src, dst, inv, w, relu):
  ks = 6 if w == 32 else 22
  f = pl.kernel(
      functools.partial(_spmm_kernel, w, relu, ks),
      out_type=jax.ShapeDtypeStruct((2 * NPAD, w), jnp.float32),
      mesh=_get_mesh(),
      compiler_params=pltpu.CompilerParams(use_tc_tiling_on_sc=False),
      scratch_types=[
          pltpu.VMEM_SHARED((NPAD, w), jnp.float32),
          pltpu.VMEM((ks, CH), jnp.int32),
          pltpu.VMEM((ks, CH), jnp.int32),
          pltpu.VMEM((ks, CH), jnp.int32),
          pltpu.VMEM((ks, CH, w), jnp.float32),
          pltpu.VMEM((CH + LN,), jnp.float32),
          pltpu.SemaphoreType.DMA,
          pltpu.SemaphoreType.DMA,
          pltpu.SemaphoreType.DMA,
      ],
  )
  return f(table, src, dst, inv)


def _head_kernel(nt0, nt1, nt2, nt3, et0, et1, et2, et3,
                 marks_hbm, emarks_hbm, h_hbm,
                 mb, eb, shft, g32, g16a, g16b, hbuf):
  c = lax.axis_index("c")
  s = lax.axis_index("s")
  wid = c * NS + s
  r0 = wid * CH
  nts = [nt0, nt1, nt2, nt3]
  ets = [et0, et1, et2, et3]

  _load_idx(marks_hbm, r0, mb)
  _load_idx(emarks_hbm, r0, eb)

  for l in range(4):
    for half in range(2):
      shift = half * NPAD
      for h in range(CH // LN):
        shft[0, pl.ds(h * LN, LN)] = mb[0, pl.ds(h * LN, LN)] + shift
      pltpu.sync_copy(nts[l].at[shft.at[0]], g32)
      col0 = 256 + l * 64 + half * 32
      def cpy(r, _, col0=col0):
        for h2 in range(2):
          hbuf[r, pl.ds(col0 + h2 * LN, LN)] = g32[r, pl.ds(h2 * LN, LN)]
        return 0
      lax.fori_loop(0, CH, cpy, 0)

  for l in range(4):
    for half in range(2):
      shift = half * NPAD
      for h in range(CH // LN):
        shft[0, pl.ds(h * LN, LN)] = eb[0, pl.ds(h * LN, LN)] + shift
      pltpu.sync_copy(ets[l].at[shft.at[0]], g16a)
      for h in range(CH // LN):
        shft[0, pl.ds(h * LN, LN)] = eb[0, pl.ds(h * LN, LN)] + (shift + 1)
      pltpu.sync_copy(ets[l].at[shft.at[0]], g16b)
      cmin = l * 32 + half * 16
      def mm(r, _, cmin=cmin):
        v1 = g16a[r, pl.ds(0, LN)]
        v2 = g16b[r, pl.ds(0, LN)]
        hbuf[r, pl.ds(cmin, LN)] = jnp.minimum(v1, v2)
        hbuf[r, pl.ds(128 + cmin, LN)] = jnp.maximum(v1, v2)
        return 0
      lax.fori_loop(0, CH, mm, 0)

  pltpu.sync_copy(hbuf, h_hbm.at[pl.ds(r0, CH), :])


@jax.jit
def _head_gather(nts, ets, marks, emarks):
  f = pl.kernel(
      _head_kernel,
      out_type=jax.ShapeDtypeStruct((4096, 512), jnp.float32),
      mesh=_get_mesh(),
      compiler_params=pltpu.CompilerParams(use_tc_tiling_on_sc=False),
      scratch_types=[
          pltpu.VMEM((1, CH), jnp.int32),
          pltpu.VMEM((1, CH), jnp.int32),
          pltpu.VMEM((1, CH), jnp.int32),
          pltpu.VMEM((CH, 32), jnp.float32),
          pltpu.VMEM((CH, 16), jnp.float32),
          pltpu.VMEM((CH, 16), jnp.float32),
          pltpu.VMEM((CH, 512), jnp.float32),
      ],
  )
  return f(*nts, *ets, marks, emarks)


BL = 512


def _mm_body(nparts, widths, *refs):
  parts = refs[:nparts]
  w_ref = refs[nparts]
  b_ref = refs[nparts + 1]
  o_ref = refs[nparts + 2]
  acc = jnp.zeros(o_ref.shape[1:], jnp.float32)
  off = 0
  for p, wp in zip(parts, widths):
    acc = acc + jnp.dot(p[...], w_ref[0, off:off + wp, :],
                        preferred_element_type=jnp.float32)
    off += wp
  o_ref[...] = (acc + b_ref[0])[None]


@functools.partial(jax.jit, static_argnums=(3,))
def _tc_mm(parts, W, b, half):
  nparts = len(parts)
  widths = tuple(p.shape[1] for p in parts)
  din = sum(widths)
  W2 = W.reshape(din, 2, half).transpose(1, 0, 2)
  b2 = b.reshape(2, 1, half)
  in_specs = [pl.BlockSpec((BL, wp), lambda i, c: (i, 0)) for wp in widths]
  in_specs.append(pl.BlockSpec((1, din, half), lambda i, c: (c, 0, 0)))
  in_specs.append(pl.BlockSpec((1, 1, half), lambda i, c: (c, 0, 0)))
  out = pl.pallas_call(
      functools.partial(_mm_body, nparts, widths),
      grid=(NPAD // BL, 2),
      in_specs=in_specs,
      out_specs=pl.BlockSpec((1, BL, half), lambda i, c: (c, i, 0)),
      out_shape=jax.ShapeDtypeStruct((2, NPAD, half), jnp.float32),
  )(*parts, W2, b2)
  return out.reshape(2 * NPAD, half)


def _mlp_body(h_ref, w1_ref, b1_ref, w2_ref, b2_ref, o_ref):
  h1 = jnp.maximum(
      jnp.dot(h_ref[...], w1_ref[...], preferred_element_type=jnp.float32)
      + b1_ref[...], 0.0)
  z = jnp.dot(h1, w2_ref[...], preferred_element_type=jnp.float32) + b2_ref[...]
  m = jnp.max(z, axis=1, keepdims=True)
  lse = m + jnp.log(jnp.sum(jnp.exp(z - m), axis=1, keepdims=True))
  o_ref[...] = z - lse


@jax.jit
def _tc_head(h, W1, b1, W2, b2):
  HB = 512
  return pl.pallas_call(
      _mlp_body,
      grid=(4096 // HB,),
      in_specs=[
          pl.BlockSpec((HB, 512), lambda i: (i, 0)),
          pl.BlockSpec((512, 128), lambda i: (0, 0)),
          pl.BlockSpec((1, 128), lambda i: (0, 0)),
          pl.BlockSpec((128, 2), lambda i: (0, 0)),
          pl.BlockSpec((1, 2), lambda i: (0, 0)),
      ],
      out_specs=pl.BlockSpec((HB, 2), lambda i: (i, 0)),
      out_shape=jax.ShapeDtypeStruct((4096, 2), jnp.float32),
  )(h, W1, b1.reshape(1, -1), W2, b2.reshape(1, -1))


def kernel(x, edge_index, marks, edge_x, edge_marks,
           Wn0, bn0, We0, be0, Wn1, bn1, We1, be1,
           Wn2, bn2, We2, be2, Wn3, bn3, We3, be3,
           W1, b1, W2, b2):
  pad_idx = jnp.full((EPAD - E_EDGES,), NPAD - 1, jnp.int32)
  ni_p = jnp.concatenate([edge_index[0], pad_idx]).reshape(-1, CH)
  ei_p = jnp.concatenate([edge_index[1], pad_idx]).reshape(-1, CH)

  dinv, binv = _degrees(ni_p, ei_p)

  x_p = jnp.zeros((NPAD, x.shape[1]), jnp.float32).at[:N_NODES].set(x)
  ex_p = jnp.zeros((NPAD, edge_x.shape[1]), jnp.float32).at[:N_NODES].set(edge_x)

  cur_parts = [x_p]
  cure_parts = [ex_p]
  node_outs, edge_outs = [], []
  Wns = [(Wn0, bn0), (Wn1, bn1), (Wn2, bn2), (Wn3, bn3)]
  Wes = [(We0, be0), (We1, be1), (We2, be2), (We3, be3)]
  for (Wn, bn), (We, be) in zip(Wns, Wes):
    xw = _tc_mm(cur_parts, Wn, bn, 32)
    ew = _tc_mm(cure_parts, We, be, 16)
    t = _spmm(xw, ni_p, ei_p, binv, 32, False)
    nout = _spmm(t, ei_p, ni_p, dinv, 32, True)
    t2 = _spmm(ew, ei_p, ni_p, dinv, 16, False)
    eout = _spmm(t2, ni_p, ei_p, binv, 16, True)
    node_outs.append(nout)
    edge_outs.append(eout)
    n3 = nout.reshape(2, NPAD, 32)
    e3 = eout.reshape(2, NPAD, 16)
    cur_parts = [n3[0], n3[1]]
    cure_parts = [e3[0], e3[1]]

  h = _head_gather(node_outs, edge_outs, marks, edge_marks)
  return _tc_head(h, W1, b1, W2, b2)


# speedup vs baseline: 1.1384x; 1.0382x over previous
import functools

import jax
import jax.numpy as jnp
from jax import lax
from jax.experimental import pallas as pl
from jax.experimental.pallas import tpu as pltpu
from jax.experimental.pallas import tpu_sc as plsc

NC = 2
NS = 16
LN = 16
CH = 128

N_NODES = 50000
ROWS_PER_TILE = 3200
NPAD = ROWS_PER_TILE * NS
E_EDGES = 800000
EDGES_PER_TILE = 50688
EPAD = EDGES_PER_TILE * NS
N_RCHUNKS = ROWS_PER_TILE // CH
CROWS_PER_TILE = EDGES_PER_TILE // CH
KD = 22
NGD = 18

@functools.cache
def _get_mesh():
  return plsc.VectorSubcoreMesh(core_axis_name="c", subcore_axis_name="s")


def _zero_chunk(buf, w):
  z = jnp.zeros((LN,), jnp.float32)
  def body(r, _):
    for h in range(w // LN):
      buf[r, pl.ds(h * LN, LN)] = z
    return 0
  lax.fori_loop(0, CH, body, 0)


def _load_idx(hbm, off, dst_row):
  pltpu.sync_copy(hbm.at[pl.ds(off, CH)], dst_row.at[0])


def _deg_body(idx_hbm, out_hbm, acc, idxb, ones, pbuf, seml, sems, s):
  def zb(k, _):
    pltpu.sync_copy(ones.at[pl.ds(CH, CH)],
                    acc.at[pl.ds(s * ROWS_PER_TILE + k * CH, CH)])
    return 0
  lax.fori_loop(0, N_RCHUNKS, zb, 0)
  plsc.subcore_barrier()

  base = s * CROWS_PER_TILE
  def eb(g, _):
    row0 = base + g * KD
    pltpu.sync_copy(idx_hbm.at[pl.ds(row0, KD), :], idxb)
    sh = [pltpu.async_copy(ones.at[pl.ds(0, CH)], acc.at[idxb.at[k]],
                           sems, add=True) for k in range(KD)]
    for h in sh:
      h.wait()
    return 0
  lax.fori_loop(0, NGD, eb, 0)
  plsc.subcore_barrier()

  def post(k, _):
    r0 = s * ROWS_PER_TILE + k * CH
    pltpu.sync_copy(acc.at[pl.ds(r0, CH)], pbuf)
    for h in range(CH // LN):
      v = pbuf[pl.ds(h * LN, LN)]
      pbuf[pl.ds(h * LN, LN)] = jnp.where(v > 0.0, 1.0 / v, 0.0)
    pltpu.sync_copy(pbuf, out_hbm.at[pl.ds(r0, CH)])
    return 0
  lax.fori_loop(0, N_RCHUNKS, post, 0)


def _deg_kernel(ni_hbm, ei_hbm, dinv_hbm, binv_hbm, acc, idxb, ones, pbuf,
                seml, sems):
  c = lax.axis_index("c")
  s = lax.axis_index("s")
  one = jnp.ones((LN,), jnp.float32)
  zero = jnp.zeros((LN,), jnp.float32)
  for h in range(CH // LN):
    ones[pl.ds(h * LN, LN)] = one
    ones[pl.ds(CH + h * LN, LN)] = zero

  @pl.when(c == 0)
  def _():
    _deg_body(ni_hbm, dinv_hbm, acc, idxb, ones, pbuf, seml, sems, s)

  @pl.when(c == 1)
  def _():
    _deg_body(ei_hbm, binv_hbm, acc, idxb, ones, pbuf, seml, sems, s)


@jax.jit
def _degrees(ni_p, ei_p):
  f = pl.kernel(
      _deg_kernel,
      out_type=[jax.ShapeDtypeStruct((NPAD,), jnp.float32),
                jax.ShapeDtypeStruct((NPAD,), jnp.float32)],
      mesh=_get_mesh(),
      compiler_params=pltpu.CompilerParams(use_tc_tiling_on_sc=False),
      scratch_types=[
          pltpu.VMEM_SHARED((NPAD,), jnp.float32),
          pltpu.VMEM((KD, CH), jnp.int32),
          pltpu.VMEM((2 * CH,), jnp.float32),
          pltpu.VMEM((CH,), jnp.float32),
          pltpu.SemaphoreType.DMA,
          pltpu.SemaphoreType.DMA,
      ],
  )
  return f(ni_p, ei_p)


def _spmm_kernel(w, relu, ks, table_hbm, src_hbm, dst_hbm, inv_hbm, out_hbm,
                 acc, sidx, didx, shft, rows, ibuf, seml, semg, sems):
  c = lax.axis_index("c")
  s = lax.axis_index("s")
  shift = c * NPAD
  ngs = CROWS_PER_TILE // ks
  pbuf = rows.at[0]

  _zero_chunk(pbuf, w)
  zh = [pltpu.async_copy(pbuf, acc.at[pl.ds(s * ROWS_PER_TILE + k * CH, CH), :],
                         semg) for k in range(N_RCHUNKS)]
  for h in zh:
    h.wait()
  plsc.subcore_barrier()

  base = s * CROWS_PER_TILE
  def eb(g, _):
    row0 = base + g * ks
    pltpu.sync_copy(src_hbm.at[pl.ds(row0, ks), :], sidx)
    lh = pltpu.async_copy(dst_hbm.at[pl.ds(row0, ks), :], didx, seml)
    for k in range(ks):
      for h in range(CH // LN):
        shft[k, pl.ds(h * LN, LN)] = sidx[k, pl.ds(h * LN, LN)] + shift
    gh = [pltpu.async_copy(table_hbm.at[shft.at[k]], rows.at[k], semg)
          for k in range(ks)]
    lh.wait()
    for h in gh:
      h.wait()
    sh = [pltpu.async_copy(rows.at[k], acc.at[didx.at[k]], sems, add=True)
          for k in range(ks)]
    for h in sh:
      h.wait()
    return 0
  lax.fori_loop(0, ngs, eb, 0)
  plsc.subcore_barrier()

  def post(k, _):
    r0 = s * ROWS_PER_TILE + k * CH
    pltpu.sync_copy(acc.at[pl.ds(r0, CH), :], pbuf)
    pltpu.sync_copy(inv_hbm.at[pl.ds(r0, CH)], ibuf.at[pl.ds(0, CH)])
    for rb in range(CH // LN):
      iv = ibuf[pl.ds(rb * LN, LN)]
      for t in range(LN):
        sv = iv[t]
        r = rb * LN + t
        for h in range(w // LN):
          v = pbuf[r, pl.ds(h * LN, LN)] * sv
          if relu:
            v = jnp.maximum(v, 0.0)
          pbuf[r, pl.ds(h * LN, LN)] = v
    pltpu.sync_copy(pbuf, out_hbm.at[pl.ds(shift + r0, CH), :])
    return 0
  lax.fori_loop(0, N_RCHUNKS, post, 0)


@functools.partial(jax.jit, static_argnums=(4, 5))
def _spmm(table, src, dst, inv, w, relu):
  ks = 6 if w == 32 else 22
  f = pl.kernel(
      functools.partial(_spmm_kernel, w, relu, ks),
      out_type=jax.ShapeDtypeStruct((2 * NPAD, w), jnp.float32),
      mesh=_get_mesh(),
      compiler_params=pltpu.CompilerParams(use_tc_tiling_on_sc=False),
      scratch_types=[
          pltpu.VMEM_SHARED((NPAD, w), jnp.float32),
          pltpu.VMEM((ks, CH), jnp.int32),
          pltpu.VMEM((ks, CH), jnp.int32),
          pltpu.VMEM((ks, CH), jnp.int32),
          pltpu.VMEM((ks, CH, w), jnp.float32),
          pltpu.VMEM((CH + LN,), jnp.float32),
          pltpu.SemaphoreType.DMA,
          pltpu.SemaphoreType.DMA,
          pltpu.SemaphoreType.DMA,
      ],
  )
  return f(table, src, dst, inv)


def _head_kernel(nt0, nt1, nt2, nt3, et0, et1, et2, et3,
                 marks_hbm, emarks_hbm, h_hbm,
                 mb, eb, shft, g32, g16a, g16b, hbuf):
  c = lax.axis_index("c")
  s = lax.axis_index("s")
  wid = c * NS + s
  r0 = wid * CH
  nts = [nt0, nt1, nt2, nt3]
  ets = [et0, et1, et2, et3]

  _load_idx(marks_hbm, r0, mb)
  _load_idx(emarks_hbm, r0, eb)

  for l in range(4):
    for half in range(2):
      shift = half * NPAD
      for h in range(CH // LN):
        shft[0, pl.ds(h * LN, LN)] = mb[0, pl.ds(h * LN, LN)] + shift
      pltpu.sync_copy(nts[l].at[shft.at[0]], g32)
      col0 = 256 + l * 64 + half * 32
      def cpy(r, _, col0=col0):
        for h2 in range(2):
          hbuf[r, pl.ds(col0 + h2 * LN, LN)] = g32[r, pl.ds(h2 * LN, LN)]
        return 0
      lax.fori_loop(0, CH, cpy, 0)

  for l in range(4):
    for half in range(2):
      shift = half * NPAD
      for h in range(CH // LN):
        shft[0, pl.ds(h * LN, LN)] = eb[0, pl.ds(h * LN, LN)] + shift
      pltpu.sync_copy(ets[l].at[shft.at[0]], g16a)
      for h in range(CH // LN):
        shft[0, pl.ds(h * LN, LN)] = eb[0, pl.ds(h * LN, LN)] + (shift + 1)
      pltpu.sync_copy(ets[l].at[shft.at[0]], g16b)
      cmin = l * 32 + half * 16
      def mm(r, _, cmin=cmin):
        v1 = g16a[r, pl.ds(0, LN)]
        v2 = g16b[r, pl.ds(0, LN)]
        hbuf[r, pl.ds(cmin, LN)] = jnp.minimum(v1, v2)
        hbuf[r, pl.ds(128 + cmin, LN)] = jnp.maximum(v1, v2)
        return 0
      lax.fori_loop(0, CH, mm, 0)

  pltpu.sync_copy(hbuf, h_hbm.at[pl.ds(r0, CH), :])


@jax.jit
def _head_gather(nts, ets, marks, emarks):
  f = pl.kernel(
      _head_kernel,
      out_type=jax.ShapeDtypeStruct((4096, 512), jnp.float32),
      mesh=_get_mesh(),
      compiler_params=pltpu.CompilerParams(use_tc_tiling_on_sc=False),
      scratch_types=[
          pltpu.VMEM((1, CH), jnp.int32),
          pltpu.VMEM((1, CH), jnp.int32),
          pltpu.VMEM((1, CH), jnp.int32),
          pltpu.VMEM((CH, 32), jnp.float32),
          pltpu.VMEM((CH, 16), jnp.float32),
          pltpu.VMEM((CH, 16), jnp.float32),
          pltpu.VMEM((CH, 512), jnp.float32),
      ],
  )
  return f(*nts, *ets, marks, emarks)


BL = 512


def _mm_body(nparts, widths, *refs):
  parts = refs[:nparts]
  w_ref = refs[nparts]
  b_ref = refs[nparts + 1]
  o_ref = refs[nparts + 2]
  acc = jnp.zeros(o_ref.shape[1:], jnp.float32)
  off = 0
  for p, wp in zip(parts, widths):
    acc = acc + jnp.dot(p[...], w_ref[0, off:off + wp, :],
                        preferred_element_type=jnp.float32)
    off += wp
  o_ref[...] = (acc + b_ref[0])[None]


@functools.partial(jax.jit, static_argnums=(3,))
def _tc_mm(parts, W, b, half):
  nparts = len(parts)
  widths = tuple(p.shape[1] for p in parts)
  din = sum(widths)
  W2 = W.reshape(din, 2, half).transpose(1, 0, 2)
  b2 = b.reshape(2, 1, half)
  in_specs = [pl.BlockSpec((BL, wp), lambda i, c: (i, 0)) for wp in widths]
  in_specs.append(pl.BlockSpec((1, din, half), lambda i, c: (c, 0, 0)))
  in_specs.append(pl.BlockSpec((1, 1, half), lambda i, c: (c, 0, 0)))
  out = pl.pallas_call(
      functools.partial(_mm_body, nparts, widths),
      grid=(NPAD // BL, 2),
      in_specs=in_specs,
      out_specs=pl.BlockSpec((1, BL, half), lambda i, c: (c, i, 0)),
      out_shape=jax.ShapeDtypeStruct((2, NPAD, half), jnp.float32),
  )(*parts, W2, b2)
  return out.reshape(2 * NPAD, half)


def _mlp_body(h_ref, w1_ref, b1_ref, w2_ref, b2_ref, o_ref):
  h1 = jnp.maximum(
      jnp.dot(h_ref[...], w1_ref[...], preferred_element_type=jnp.float32)
      + b1_ref[...], 0.0)
  z = jnp.dot(h1, w2_ref[...], preferred_element_type=jnp.float32) + b2_ref[...]
  m = jnp.max(z, axis=1, keepdims=True)
  lse = m + jnp.log(jnp.sum(jnp.exp(z - m), axis=1, keepdims=True))
  o_ref[...] = z - lse


@jax.jit
def _tc_head(h, W1, b1, W2, b2):
  HB = 512
  return pl.pallas_call(
      _mlp_body,
      grid=(4096 // HB,),
      in_specs=[
          pl.BlockSpec((HB, 512), lambda i: (i, 0)),
          pl.BlockSpec((512, 128), lambda i: (0, 0)),
          pl.BlockSpec((1, 128), lambda i: (0, 0)),
          pl.BlockSpec((128, 2), lambda i: (0, 0)),
          pl.BlockSpec((1, 2), lambda i: (0, 0)),
      ],
      out_specs=pl.BlockSpec((HB, 2), lambda i: (i, 0)),
      out_shape=jax.ShapeDtypeStruct((4096, 2), jnp.float32),
  )(h, W1, b1.reshape(1, -1), W2, b2.reshape(1, -1))


def kernel(x, edge_index, marks, edge_x, edge_marks,
           Wn0, bn0, We0, be0, Wn1, bn1, We1, be1,
           Wn2, bn2, We2, be2, Wn3, bn3, We3, be3,
           W1, b1, W2, b2):
  pad_idx = jnp.full((EPAD - E_EDGES,), NPAD - 1, jnp.int32)
  ni_p = jnp.concatenate([edge_index[0], pad_idx]).reshape(-1, CH)
  ei_p = jnp.concatenate([edge_index[1], pad_idx]).reshape(-1, CH)

  dinv, binv = _degrees(ni_p, ei_p)

  x_p = jnp.zeros((NPAD, x.shape[1]), jnp.float32).at[:N_NODES].set(x)
  ex_p = jnp.zeros((NPAD, edge_x.shape[1]), jnp.float32).at[:N_NODES].set(edge_x)

  cur_parts = [x_p]
  cure_parts = [ex_p]
  node_outs, edge_outs = [], []
  Wns = [(Wn0, bn0), (Wn1, bn1), (Wn2, bn2), (Wn3, bn3)]
  Wes = [(We0, be0), (We1, be1), (We2, be2), (We3, be3)]
  for (Wn, bn), (We, be) in zip(Wns, Wes):
    xw = _tc_mm(cur_parts, Wn, bn, 32)
    ew = _tc_mm(cure_parts, We, be, 16)
    t = _spmm(xw, ni_p, ei_p, binv, 32, False)
    nout = _spmm(t, ei_p, ni_p, dinv, 32, True)
    t2 = _spmm(ew, ei_p, ni_p, dinv, 16, False)
    eout = _spmm(t2, ni_p, ei_p, binv, 16, True)
    node_outs.append(nout)
    edge_outs.append(eout)
    n3 = nout.reshape(2, NPAD, 32)
    e3 = eout.reshape(2, NPAD, 16)
    cur_parts = [n3[0], n3[1]]
    cure_parts = [e3[0], e3[1]]

  h = _head_gather(node_outs, edge_outs, marks, edge_marks)
  return _tc_head(h, W1, b1, W2, b2)
